# baseline (speedup 1.0000x reference)
import functools

import jax
import jax.numpy as jnp
from jax import lax
from jax.experimental import pallas as pl
from jax.experimental.pallas import tpu as pltpu
from jax.experimental.pallas import tpu_sc as plsc

B = 16384
F = 26
D = 32
V = 100000

NW = 32
BPW = B // NW
CB = 8
NCHUNK = BPW // CB
ROWS = CB * F

_mesh = plsc.VectorSubcoreMesh(core_axis_name="c", subcore_axis_name="s")


@functools.partial(
    pl.kernel,
    out_type=jax.ShapeDtypeStruct((B, F * D), jnp.float32),
    mesh=_mesh,
    scratch_types=[
        pltpu.VMEM((6, 4 * BPW), jnp.int32),
        pltpu.VMEM((2 * BPW,), jnp.int32),
        pltpu.VMEM((ROWS, 128), jnp.float32),
        pltpu.VMEM((CB, F * D), jnp.float32),
        pltpu.SemaphoreType.DMA,
    ],
    compiler_params=pltpu.CompilerParams(use_tc_tiling_on_sc=True),
)
def _sc_gather(xg6_hbm, xg2_hbm, w_hbm, w2_hbm, out_hbm,
               xv6, xv2, gbuf, outbuf, sem):
    wid = lax.axis_index("s") * 2 + lax.axis_index("c")
    pltpu.sync_copy(xg6_hbm.at[:, pl.ds(wid * 4 * BPW, 4 * BPW)], xv6)
    pltpu.sync_copy(xg2_hbm.at[pl.ds(wid * 2 * BPW, 2 * BPW)], xv2)

    def chunk_body(c, carry):
        copies = []
        for w in range(6):
            copies.append(pltpu.async_copy(
                w_hbm.at[xv6.at[w, pl.ds(c * 4 * CB, 4 * CB)],
                         pl.ds(128 * w, 128)],
                gbuf.at[pl.ds(32 * w, 32)], sem))
        copies.append(pltpu.async_copy(
            w2_hbm.at[xv2.at[pl.ds(c * 2 * CB, 2 * CB)]],
            gbuf.at[pl.ds(192, 16)], sem))
        for cp in copies:
            cp.wait()

        def ext_body(b, _):
            for f in range(24):
                src = 32 * (f // 4) + b * 4 + (f % 4)
                off = 32 * (f % 4)
                outbuf[b, pl.ds(32 * f, 16)] = gbuf[src, pl.ds(off, 16)]
                outbuf[b, pl.ds(32 * f + 16, 16)] = gbuf[src, pl.ds(off + 16, 16)]
            for f in range(24, F):
                src = 192 + b * 2 + (f - 24)
                off = 32 * (f - 24)
                outbuf[b, pl.ds(32 * f, 16)] = gbuf[src, pl.ds(off, 16)]
                outbuf[b, pl.ds(32 * f + 16, 16)] = gbuf[src, pl.ds(off + 16, 16)]
            return _

        lax.fori_loop(0, CB, ext_body, None)
        pltpu.sync_copy(outbuf, out_hbm.at[pl.ds(wid * BPW + c * CB, CB)])
        return carry

    lax.fori_loop(0, NCHUNK, chunk_body, None)


def kernel(x, weight):
    x32 = x.astype(jnp.int32)
    xg6 = x32[:, :24].reshape(B, 6, 4).transpose(1, 0, 2).reshape(6, 4 * B)
    xg2 = x32[:, 24:].reshape(2 * B)
    w2 = jnp.pad(lax.slice(weight, (0, 768), (V, 832)), ((0, 0), (0, 64)))
    return _sc_gather(xg6, xg2, weight, w2)
